# baseline (speedup 1.0000x reference)
import jax
import jax.numpy as jnp
from jax.experimental import pallas as pl
from jax.experimental.pallas import tpu as pltpu

_N = 10000
_DF = 128
_DH = 64
_DE = 16
_B = 400
_NB = _N // _B
_B3 = 200


def _k1(feats_ref, w1_ref, w2_ref, adj_ref, z_ref, x1_ref, y_ref):
    i = pl.program_id(0)

    @pl.when(i == 0)
    def _():
        x1_ref[...] = jnp.dot(feats_ref[...], w1_ref[...],
                              preferred_element_type=jnp.float32)

    @pl.when(i < _NB)
    def _():
        h = jnp.dot(adj_ref[...], x1_ref[...],
                    preferred_element_type=jnp.float32)
        h = jnp.maximum(h, 0.0)
        y_ref[pl.ds(i * _B, _B), :] = jnp.dot(
            h, w2_ref[...], preferred_element_type=jnp.float32)

    @pl.when(i >= _NB)
    def _():
        z = jnp.dot(adj_ref[...], y_ref[...],
                    preferred_element_type=jnp.float32)
        z_ref[...] = jnp.maximum(z, 0.0)


def _k2(zi_ref, zall_ref, adj_ref, out_ref):
    out_ref[...] = jax.lax.dot_general(
        zi_ref[...], zall_ref[...],
        (((1,), (1,)), ((), ())),
        preferred_element_type=jnp.float32) + adj_ref[0, 0] * 0.0


def kernel(feats, adj, W1, W2):
    z = pl.pallas_call(
        _k1,
        grid=(2 * _NB,),
        in_specs=[
            pl.BlockSpec((_N, _DF), lambda i: (0, 0)),
            pl.BlockSpec((_DF, _DH), lambda i: (0, 0)),
            pl.BlockSpec((_DH, _DE), lambda i: (0, 0)),
            pl.BlockSpec((_B, _N), lambda i: (jax.lax.rem(i, _NB), 0)),
        ],
        out_specs=pl.BlockSpec((_B, _DE), lambda i: (jnp.maximum(i - _NB, 0), 0)),
        out_shape=jax.ShapeDtypeStruct((_N, _DE), jnp.float32),
        scratch_shapes=[
            pltpu.VMEM((_N, _DH), jnp.float32),
            pltpu.VMEM((_N, _DE), jnp.float32),
        ],
        compiler_params=pltpu.CompilerParams(
            dimension_semantics=("arbitrary",)),
    )(feats, W1, W2, adj)

    nb3 = _N // _B3
    out = pl.pallas_call(
        _k2,
        grid=(nb3,),
        in_specs=[
            pl.BlockSpec((_B3, _DE), lambda i: (i, 0)),
            pl.BlockSpec((_N, _DE), lambda i: (0, 0)),
            pl.BlockSpec((_B3, _N), lambda i: (i, 0)),
        ],
        out_specs=pl.BlockSpec((_B3, _N), lambda i: (i, 0)),
        out_shape=jax.ShapeDtypeStruct((_N, _N), jnp.float32),
        compiler_params=pltpu.CompilerParams(
            dimension_semantics=("parallel",)),
    )(z, z, adj)
    return out


# speedup vs baseline: 1.3595x; 1.3595x over previous
import jax
import jax.numpy as jnp
from jax.experimental import pallas as pl
from jax.experimental.pallas import tpu as pltpu

_N = 10000
_DF = 128
_DH = 64
_DE = 16
_B = 200
_NB = _N // _B
_B3 = 200
_NB3 = _N // _B3


def _fused(feats_ref, w1_ref, w2_ref, adj_ref, out_ref, x1_ref, y_ref, z_ref):
    i = pl.program_id(0)

    @pl.when(i == 0)
    def _():
        x1_ref[...] = jnp.dot(feats_ref[...], w1_ref[...],
                              preferred_element_type=jnp.float32)

    @pl.when(i < _NB)
    def _():
        h = jnp.dot(adj_ref[...], x1_ref[...],
                    preferred_element_type=jnp.float32)
        h = jnp.maximum(h, 0.0)
        y_ref[pl.ds(i * _B, _B), :] = jnp.dot(
            h, w2_ref[...], preferred_element_type=jnp.float32)

    @pl.when(jnp.logical_and(i >= _NB, i < 2 * _NB))
    def _():
        z = jnp.dot(adj_ref[...], y_ref[...],
                    preferred_element_type=jnp.float32)
        z_ref[pl.ds((i - _NB) * _B, _B), :] = jnp.maximum(z, 0.0)

    @pl.when(i >= 2 * _NB)
    def _():
        b = i - 2 * _NB
        out_ref[...] = jax.lax.dot_general(
            z_ref[pl.ds(b * _B3, _B3), :], z_ref[...],
            (((1,), (1,)), ((), ())),
            preferred_element_type=jnp.float32)


def kernel(feats, adj, W1, W2):
    out = pl.pallas_call(
        _fused,
        grid=(2 * _NB + _NB3,),
        in_specs=[
            pl.BlockSpec((_N, _DF), lambda i: (0, 0)),
            pl.BlockSpec((_DF, _DH), lambda i: (0, 0)),
            pl.BlockSpec((_DH, _DE), lambda i: (0, 0)),
            pl.BlockSpec((_B, _N),
                         lambda i: (jnp.where(i < 2 * _NB,
                                              jax.lax.rem(i, _NB),
                                              _NB - 1), 0)),
        ],
        out_specs=pl.BlockSpec((_B3, _N),
                               lambda i: (jnp.maximum(i - 2 * _NB, 0), 0)),
        out_shape=jax.ShapeDtypeStruct((_N, _N), jnp.float32),
        scratch_shapes=[
            pltpu.VMEM((_N, _DH), jnp.float32),
            pltpu.VMEM((_N, _DE), jnp.float32),
            pltpu.VMEM((_N, _DE), jnp.float32),
        ],
        compiler_params=pltpu.CompilerParams(
            dimension_semantics=("arbitrary",)),
    )(feats, W1, W2, adj)
    return out
